# baseline (speedup 1.0000x reference)
import jax
import jax.numpy as jnp
from jax.experimental import pallas as pl
from jax.experimental.pallas import tpu as pltpu

_N = 10000
_C = 128
_K = 4
_BLOCK = 200
_NEG = float("-inf")


def _mb_kernel(mem_ref, r_ref, out_d_ref, out_i_ref, memn_ref, slab_ref):
    i = pl.program_id(0)

    @pl.when(i == 0)
    def _():
        x = mem_ref[...]
        norms = jnp.sqrt(jnp.sum(x * x, axis=1, keepdims=True))
        memn_ref[...] = x / jnp.maximum(norms, 1e-12)

    block = memn_ref[pl.ds(i * _BLOCK, _BLOCK), :]
    iota = jax.lax.broadcasted_iota(jnp.int32, (_BLOCK, _N), 1)
    r = r_ref[...]

    diagidx = i * _BLOCK + jax.lax.broadcasted_iota(
        jnp.int32, (_BLOCK, 1), 0)
    diagv = jnp.sum(block * block, axis=1, keepdims=True)

    sims = jax.lax.dot_general(
        block, memn_ref[...], (((1,), (1,)), ((), ())),
        preferred_element_type=jnp.float32)
    slab_ref[...] = jnp.where(iota == diagidx, _NEG, sims)

    sel_d = jnp.zeros((_BLOCK, 1), jnp.float32)
    sel_i = jnp.zeros((_BLOCK, 1), jnp.int32)
    ms, idxs = [], []
    for rank in range(1, _K):
        s = slab_ref[...]
        m = jnp.max(s, axis=1, keepdims=True)
        idx = jnp.argmax(s, axis=1).astype(jnp.int32)[:, None]
        ms.append(m)
        idxs.append(idx)
        hit = r == rank
        sel_d = jnp.where(hit, m, sel_d)
        sel_i = jnp.where(hit, idx, sel_i)
        if rank < _K - 1:
            slab_ref[...] = jnp.where(iota == idx, _NEG, s)
    out_d_ref[...] = sel_d
    out_i_ref[...] = sel_i

    @pl.when(jnp.any(ms[0] >= diagv - 1e-4))
    def _():
        slab_ref[...] = jnp.where(iota == idxs[0], ms[0], slab_ref[...])
        slab_ref[...] = jnp.where(iota == idxs[1], ms[1], slab_ref[...])
        slab_ref[...] = jnp.where(iota == diagidx, diagv, slab_ref[...])
        f_d = jnp.zeros((_BLOCK, 1), jnp.float32)
        f_i = jnp.zeros((_BLOCK, 1), jnp.int32)
        for rank in range(_K):
            sf = slab_ref[...]
            m = jnp.max(sf, axis=1, keepdims=True)
            idx = jnp.argmax(sf, axis=1).astype(jnp.int32)[:, None]
            if rank >= 1:
                hit = r == rank
                f_d = jnp.where(hit, m, f_d)
                f_i = jnp.where(hit, idx, f_i)
            if rank < _K - 1:
                slab_ref[...] = jnp.where(iota == idx, _NEG, sf)
        out_d_ref[...] = f_d
        out_i_ref[...] = f_i


def kernel(memory, randk, topk):
    n = memory.shape[0]
    nb = n // _BLOCK
    r = (randk + topk - 2).astype(jnp.int32).reshape(n, 1)
    sel_d, sel_i = pl.pallas_call(
        _mb_kernel,
        grid=(nb,),
        in_specs=[
            pl.BlockSpec((n, _C), lambda i: (0, 0)),
            pl.BlockSpec((_BLOCK, 1), lambda i: (i, 0)),
        ],
        out_specs=[
            pl.BlockSpec((_BLOCK, 1), lambda i: (i, 0)),
            pl.BlockSpec((_BLOCK, 1), lambda i: (i, 0)),
        ],
        out_shape=[
            jax.ShapeDtypeStruct((n, 1), jnp.float32),
            jax.ShapeDtypeStruct((n, 1), jnp.int32),
        ],
        scratch_shapes=[
            pltpu.VMEM((n, _C), jnp.float32),
            pltpu.VMEM((_BLOCK, _N), jnp.float32),
        ],
        compiler_params=pltpu.CompilerParams(
            dimension_semantics=("arbitrary",)),
    )(memory, r)
    return sel_d.reshape(n), sel_i.reshape(n)


# speedup vs baseline: 1.1328x; 1.1328x over previous
import jax
import jax.numpy as jnp
from jax.experimental import pallas as pl
from jax.experimental.pallas import tpu as pltpu

_N = 10000
_C = 128
_K = 4
_BLOCK = 400
_NEG = float("-inf")


def _norm_kernel(mem_ref, memn_ref):
    x = mem_ref[...]
    norms = jnp.sqrt(jnp.sum(x * x, axis=1, keepdims=True))
    memn_ref[...] = x / jnp.maximum(norms, 1e-12)


def _mb_kernel(memn_ref, r_ref, out_d_ref, out_i_ref):
    i = pl.program_id(0)

    block = memn_ref[pl.ds(i * _BLOCK, _BLOCK), :]
    sims = jax.lax.dot_general(
        block, memn_ref[...], (((1,), (1,)), ((), ())),
        preferred_element_type=jnp.float32)

    iota = jax.lax.broadcasted_iota(jnp.int32, (_BLOCK, _N), 1)
    r = r_ref[...]

    diagidx = i * _BLOCK + jax.lax.broadcasted_iota(
        jnp.int32, (_BLOCK, 1), 0)
    diagv = jnp.sum(block * block, axis=1, keepdims=True)
    s = jnp.where(iota == diagidx, _NEG, sims)

    sel_d = jnp.zeros((_BLOCK, 1), jnp.float32)
    sel_i = jnp.zeros((_BLOCK, 1), jnp.int32)
    m1 = None
    for rank in range(1, _K):
        m = jnp.max(s, axis=1, keepdims=True)
        if rank == 1:
            m1 = m
        idx = jnp.argmax(s, axis=1).astype(jnp.int32)[:, None]
        hit = r == rank
        sel_d = jnp.where(hit, m, sel_d)
        sel_i = jnp.where(hit, idx, sel_i)
        if rank < _K - 1:
            s = jnp.where(iota == idx, _NEG, s)
    out_d_ref[...] = sel_d
    out_i_ref[...] = sel_i

    @pl.when(jnp.any(m1 >= diagv - 1e-4))
    def _():
        sf = jax.lax.dot_general(
            block, memn_ref[...], (((1,), (1,)), ((), ())),
            preferred_element_type=jnp.float32)
        fiota = jax.lax.broadcasted_iota(jnp.int32, (_BLOCK, _N), 1)
        f_d = jnp.zeros((_BLOCK, 1), jnp.float32)
        f_i = jnp.zeros((_BLOCK, 1), jnp.int32)
        for rank in range(_K):
            m = jnp.max(sf, axis=1, keepdims=True)
            idx = jnp.argmax(sf, axis=1).astype(jnp.int32)[:, None]
            if rank >= 1:
                hit = r == rank
                f_d = jnp.where(hit, m, f_d)
                f_i = jnp.where(hit, idx, f_i)
            if rank < _K - 1:
                sf = jnp.where(fiota == idx, _NEG, sf)
        out_d_ref[...] = f_d
        out_i_ref[...] = f_i


def kernel(memory, randk, topk):
    n = memory.shape[0]
    nb = n // _BLOCK
    memn = pl.pallas_call(
        _norm_kernel,
        out_shape=jax.ShapeDtypeStruct((n, _C), jnp.float32),
    )(memory)
    r = (randk + topk - 2).astype(jnp.int32).reshape(n, 1)
    sel_d, sel_i = pl.pallas_call(
        _mb_kernel,
        grid=(nb,),
        in_specs=[
            pl.BlockSpec((n, _C), lambda i: (0, 0)),
            pl.BlockSpec((_BLOCK, 1), lambda i: (i, 0)),
        ],
        out_specs=[
            pl.BlockSpec((_BLOCK, 1), lambda i: (i, 0)),
            pl.BlockSpec((_BLOCK, 1), lambda i: (i, 0)),
        ],
        out_shape=[
            jax.ShapeDtypeStruct((n, 1), jnp.float32),
            jax.ShapeDtypeStruct((n, 1), jnp.int32),
        ],
        compiler_params=pltpu.CompilerParams(
            dimension_semantics=("arbitrary",),
            vmem_limit_bytes=100 * 1024 * 1024),
    )(memn, r)
    return sel_d.reshape(n), sel_i.reshape(n)
